# Initial kernel scaffold; baseline (speedup 1.0000x reference)
import jax
import jax.numpy as jnp
from jax.experimental import pallas as pl


def kernel(embeddings, instance_ids):
    raise NotImplementedError("write your pallas kernel here")


# speedup vs baseline: 20.0035x; 20.0035x over previous
import functools

import jax
import jax.numpy as jnp
from jax import lax
from jax.experimental import pallas as pl
from jax.experimental.pallas import tpu as pltpu
from jax.experimental.pallas import tpu_sc as plsc

NB = 8
N = 32768
D = 16
K = 64
NC, NS, L = 2, 16, 16
QUARTER = N // 4
CHUNK = 2048
NCHUNK = QUARTER // CHUNK
NGRP = CHUNK // L

DELTA_V = 0.5
DELTA_D = 1.5


def _iota():
    return lax.iota(jnp.int32, L)


def _splat_i(x):
    return jnp.full((L,), x, jnp.int32)


def _splat_f(x):
    return jnp.full((L,), x, jnp.float32)


def _sqrt16(s):
    i = plsc.bitcast(s, jnp.int32)
    i = jnp.int32(0x5F3759DF) - lax.shift_right_logical(i, 1)
    y = plsc.bitcast(i, jnp.float32)
    for _ in range(3):
        y = y * (1.5 - 0.5 * s * y * y)
    return s * y


def _body(emb_hbm, ids_hbm, out_hbm,
          ids_v, ebuf0, ebuf1, bnd_v, sums_v, gsum_v, tmp_a, tmp_b,
          tmp_c, tmp_d, cinv_v, ovec_v, shared_spmem, sem0, sem1):
    c = lax.axis_index("c")
    s = lax.axis_index("s")
    b = c * 4 + s // 4
    q = s % 4
    qbase = q * QUARTER
    iota = _iota()

    ebufs = (ebuf0, ebuf1)
    sems = (sem0, sem1)

    def start(chunk):
        return pltpu.async_copy(
            emb_hbm.at[b, pl.ds(qbase + chunk * CHUNK, CHUNK)],
            ebufs[chunk % 2], sems[chunk % 2])

    dma = start(0)
    pltpu.sync_copy(ids_hbm.at[b, pl.ds(qbase, QUARTER)], ids_v)

    for grp in range(5):
        kv = iota + _splat_i(grp * L)
        lo = _splat_i(0)
        hi = _splat_i(QUARTER)
        for _ in range(13):
            mid = lax.shift_right_logical(lo + hi, 1)
            v = plsc.load_gather(ids_v, [mid])
            pred = v < kv
            lo = jnp.where(pred, mid + 1, lo)
            hi = jnp.where(pred, hi, mid)
        bnd_v[pl.ds(grp * L, L)] = lo

    zf = _splat_f(0.0)
    for k in range(K):
        sums_v[k, :] = zf
    for m in range(4):
        cnt = (bnd_v[pl.ds(m * L + 1, L)] - bnd_v[pl.ds(m * L, L)])
        sums_v[K + m, :] = cnt.astype(jnp.float32)

    def pass1_chunk(buf, cbase):
        def k_body(k, _):
            ksp = _splat_i(k)
            blo = plsc.load_gather(bnd_v, [ksp])
            bhi = plsc.load_gather(bnd_v, [ksp + 1])
            lo = jnp.max(jnp.clip(blo - cbase, 0, CHUNK))
            hi = jnp.max(jnp.clip(bhi - cbase, 0, CHUNK))

            def n_body(n, acc):
                return acc + plsc.load_gather(buf, [_splat_i(n), iota])

            acc = lax.fori_loop(lo, hi, n_body, zf)
            plsc.addupdate_scatter(sums_v, [ksp, iota], acc)
            return 0

        lax.fori_loop(0, K, k_body, 0)

    for chunk in range(NCHUNK):
        nxt = start(chunk + 1) if chunk + 1 < NCHUNK else None
        dma.wait()
        pass1_chunk(ebufs[chunk % 2], chunk * CHUNK)
        dma = nxt

    dma = start(0)

    pltpu.sync_copy(sums_v, shared_spmem.at[s])
    plsc.subcore_barrier()
    base = (s // 4) * 4
    tmps = (tmp_a, tmp_b, tmp_c, tmp_d)
    for j in range(4):
        pltpu.sync_copy(shared_spmem.at[base + j], tmps[j])
    for r in range(K + 4):
        acc = (tmp_a[r, :] + tmp_b[r, :] + tmp_c[r, :] + tmp_d[r, :])
        gsum_v[r, :] = acc
    for m in range(4):
        cinv_v[pl.ds(m * L, L)] = 1.0 / jnp.maximum(gsum_v[K + m, :], 1.0)

    def pass2_chunk(buf, cbase, vacc):
        def g_body(g, vacc):
            n0 = g * L
            pvec = iota + _splat_i(n0)
            idv = ids_v[pl.ds(cbase + n0, L)]
            civ = plsc.load_gather(cinv_v, [idv])
            sacc = _splat_f(1e-12)
            for d in range(D):
                e_d = plsc.load_gather(buf, [pvec, _splat_i(d)])
                s_d = plsc.load_gather(gsum_v, [idv, _splat_i(d)])
                diff = e_d - s_d * civ
                sacc = sacc + diff * diff
            dist = _sqrt16(sacc)
            hin = jnp.maximum(dist - DELTA_V, 0.0)
            return vacc + hin * hin * civ

        return lax.fori_loop(0, NGRP, g_body, vacc)

    vacc = zf
    for chunk in range(NCHUNK):
        nxt = start(chunk + 1) if chunk + 1 < NCHUNK else None
        dma.wait()
        vacc = pass2_chunk(ebufs[chunk % 2], chunk * CHUNK, vacc)
        dma = nxt

    ivec = iota + _splat_i(16 * (s % 4))
    civ_i = plsc.load_gather(cinv_v, [ivec])
    mi = []
    rsq = _splat_f(1e-12)
    for d in range(D):
        md = plsc.load_gather(gsum_v, [ivec, _splat_i(d)]) * civ_i
        mi.append(md)
        rsq = rsq + md * md
    racc = _sqrt16(rsq)

    def j_body(j, dacc):
        jv = _splat_i(j)
        cj = plsc.load_gather(cinv_v, [jv])
        sq = _splat_f(0.0)
        for d in range(D):
            s_jd = plsc.load_gather(gsum_v, [jv, _splat_i(d)])
            diff = mi[d] - s_jd * cj
            sq = sq + diff * diff
        pd = _sqrt16(sq)
        h = jnp.maximum(2.0 * DELTA_D - pd, 0.0)
        h = h * h
        h = jnp.where(ivec == j, 0.0, h)
        return dacc + h

    dacc = lax.fori_loop(0, K, j_body, zf)

    ovec_v[0, :] = vacc
    ovec_v[1, :] = dacc
    ovec_v[2, :] = racc
    pltpu.sync_copy(ovec_v, out_hbm.at[c * NS + s])


@jax.jit
def _sc_partials(embeddings, instance_ids):
    mesh = plsc.VectorSubcoreMesh(core_axis_name="c", subcore_axis_name="s",
                                  num_cores=NC, num_subcores=NS)
    return pl.kernel(
        _body,
        out_type=jax.ShapeDtypeStruct((NC * NS, 3, L), jnp.float32),
        mesh=mesh,
        compiler_params=pltpu.CompilerParams(needs_layout_passes=False,
                                             use_tc_tiling_on_sc=False),
        scratch_types=[
            pltpu.VMEM((QUARTER,), jnp.int32),
            pltpu.VMEM((CHUNK, D), jnp.float32),
            pltpu.VMEM((CHUNK, D), jnp.float32),
            pltpu.VMEM((80,), jnp.int32),
            pltpu.VMEM((K + 4, L), jnp.float32),
            pltpu.VMEM((K + 4, L), jnp.float32),
            pltpu.VMEM((K + 4, L), jnp.float32),
            pltpu.VMEM((K + 4, L), jnp.float32),
            pltpu.VMEM((K + 4, L), jnp.float32),
            pltpu.VMEM((K + 4, L), jnp.float32),
            pltpu.VMEM((K,), jnp.float32),
            pltpu.VMEM((3, L), jnp.float32),
            pltpu.VMEM_SHARED((NS, K + 4, L), jnp.float32),
            pltpu.SemaphoreType.DMA,
            pltpu.SemaphoreType.DMA,
        ],
    )(embeddings, instance_ids)


def kernel(embeddings, instance_ids):
    part = _sc_partials(embeddings, instance_ids.astype(jnp.int32))
    r = part.reshape(NC, 4, 4, 3, L).sum(axis=(2, 4))
    r = r.reshape(NB, 3)
    num_pairs = K * (K - 1) / 2.0
    var_b = r[:, 0] / K
    dist_b = r[:, 1] / (2.0 * num_pairs)
    reg_b = r[:, 2] / K
    var_loss = jnp.mean(var_b)
    dist_loss = jnp.mean(dist_b)
    reg_loss = jnp.mean(reg_b)
    total = var_loss + dist_loss + 0.001 * reg_loss
    return (total, var_loss, dist_loss, reg_loss)
